# Initial kernel scaffold; baseline (speedup 1.0000x reference)
import jax
import jax.numpy as jnp
from jax.experimental import pallas as pl
from jax.experimental.pallas import tpu as pltpu

B = 4096
D = 256
E = 16
K = 4
DFF = 512
BLK = 512
NBLK = B // BLK

_INTERPRET = False


def _dot(a, b):
    return jnp.dot(a, b, preferred_element_type=jnp.float32)


def _topk_gates(logits):
    l = logits
    iota = jax.lax.broadcasted_iota(jnp.int32, l.shape, 1)
    tvs, tis = [], []
    for _ in range(K):
        m = jnp.max(l, axis=1, keepdims=True)
        idx = jnp.min(jnp.where(l == m, iota, E), axis=1, keepdims=True)
        tvs.append(m)
        tis.append(idx)
        l = jnp.where(iota == idx, -jnp.inf, l)
    topv = jnp.concatenate(tvs, axis=1)
    topi = jnp.concatenate(tis, axis=1)
    g = jnp.exp(topv - topv[:, 0:1])
    gates = g / jnp.sum(g, axis=1, keepdims=True)
    comb = jnp.zeros_like(logits)
    cnt = jnp.zeros_like(logits)
    for k in range(K):
        sel = iota == tis[k]
        comb = comb + jnp.where(sel, gates[:, k:k + 1], 0.0)
        cnt = cnt + jnp.where(sel, 1.0, 0.0)
    return topi, comb, cnt


def _router_stats(logits, comb_cnt, imp_ref, cnt_ref):
    mx = jnp.max(logits, axis=1, keepdims=True)
    ex = jnp.exp(logits - mx)
    sm = ex / jnp.sum(ex, axis=1, keepdims=True)

    @pl.when(pl.program_id(0) == 0)
    def _():
        imp_ref[0, 0, :] = jnp.zeros((E,), jnp.float32)
        cnt_ref[0, 0, :] = jnp.zeros((E,), jnp.float32)

    imp_ref[0, 0, :] += jnp.sum(sm, axis=0)
    cnt_ref[0, 0, :] += jnp.sum(comb_cnt, axis=0)


def _ffn_dense(x, comb, W1_ref, b1_ref, W2_ref, b2_ref):
    out = jnp.zeros_like(x)
    for e in range(E):
        h = jnp.maximum(_dot(x, W1_ref[e]) + b1_ref[e:e + 1, :], 0.0)
        y = _dot(h, W2_ref[e]) + b2_ref[e:e + 1, :]
        out = out + comb[:, e:e + 1] * y
    return x + out


def _stage0_body(opi_ref, a_ref, b_ref, c_ref, opt_ref, Wp_ref, bp_ref,
                 Wr_ref, br_ref, W1_ref, b1_ref, W2_ref, b2_ref,
                 x1_ref, imp_ref, cnt_ref):
    op = opi_ref[0, 0, :][:, None]
    av = a_ref[0, 0, :][:, None]
    bv = b_ref[0, 0, :][:, None]
    cv = c_ref[0, 0, :][:, None]
    i8 = jax.lax.broadcasted_iota(jnp.int32, (BLK, 8), 1)
    onehot = (op == i8).astype(jnp.float32)
    abits = ((av >> i8) & 1).astype(jnp.float32)
    bbits = ((bv >> i8) & 1).astype(jnp.float32)
    cf = cv.astype(jnp.float32)
    P = _dot(opt_ref[...], Wp_ref[0:32, :])
    x = (_dot(onehot, P)
         + _dot(abits, Wp_ref[32:40, :])
         + _dot(bbits, Wp_ref[40:48, :])
         + cf * Wp_ref[48:49, :]
         + bp_ref[...])
    logits = _dot(x, Wr_ref[...]) + br_ref[...]
    _, comb, cnt_rows = _topk_gates(logits)
    _router_stats(logits, cnt_rows, imp_ref, cnt_ref)
    x1_ref[...] = _ffn_dense(x, comb, W1_ref, b1_ref, W2_ref, b2_ref)


def _stage1_body(x_ref, Wr_ref, br_ref, W1_ref, b1_ref, W2_ref, b2_ref,
                 H1_ref, bh1_ref, H2_ref, bh2_ref,
                 res_ref, topi_ref, imp_ref, cnt_ref):
    x = x_ref[...]
    logits = _dot(x, Wr_ref[...]) + br_ref[...]
    topi, comb, cnt_rows = _topk_gates(logits)
    _router_stats(logits, cnt_rows, imp_ref, cnt_ref)
    x2 = _ffn_dense(x, comb, W1_ref, b1_ref, W2_ref, b2_ref)
    h = jnp.maximum(_dot(x2, H1_ref[...]) + bh1_ref[...], 0.0)
    z = _dot(h, H2_ref[...]) + bh2_ref[...]
    res_ref[...] = 1.0 / (1.0 + jnp.exp(-z))
    topi_ref[...] = topi


def _full(shape):
    nd = len(shape)
    return pl.BlockSpec(shape, lambda i: (0,) * nd)


def kernel(op_idx, a, b, c, op_table, Wp, bp, Wr, br, W1, b1, W2, b2, H1, bh1, H2, bh2):
    tok3 = lambda v: v.reshape(NBLK, 1, BLK)
    tokspec = pl.BlockSpec((1, 1, BLK), lambda i: (i, 0, 0))
    accspec = pl.BlockSpec((1, 1, E), lambda i: (0, 0, 0))
    accshape = jax.ShapeDtypeStruct((1, 1, E), jnp.float32)

    x1, imp0, cnt0 = pl.pallas_call(
        _stage0_body,
        grid=(NBLK,),
        in_specs=[tokspec, tokspec, tokspec, tokspec,
                  _full((8, 32)), _full((49, D)), _full((1, D)),
                  _full((D, E)), _full((1, E)),
                  _full((E, D, DFF)), _full((E, DFF)),
                  _full((E, DFF, D)), _full((E, D))],
        out_specs=[pl.BlockSpec((BLK, D), lambda i: (i, 0)), accspec, accspec],
        out_shape=[jax.ShapeDtypeStruct((B, D), jnp.float32), accshape, accshape],
        interpret=_INTERPRET,
    )(tok3(op_idx), tok3(a), tok3(b), tok3(c), op_table, Wp, bp[None, :],
      Wr[0], br[0][None, :], W1[0], b1[0], W2[0], b2[0])

    res, topi, imp1, cnt1 = pl.pallas_call(
        _stage1_body,
        grid=(NBLK,),
        in_specs=[pl.BlockSpec((BLK, D), lambda i: (i, 0)),
                  _full((D, E)), _full((1, E)),
                  _full((E, D, DFF)), _full((E, DFF)),
                  _full((E, DFF, D)), _full((E, D)),
                  _full((D, 64)), _full((1, 64)), _full((64, 8)), _full((1, 8))],
        out_specs=[pl.BlockSpec((BLK, 8), lambda i: (i, 0)),
                   pl.BlockSpec((BLK, K), lambda i: (i, 0)),
                   accspec, accspec],
        out_shape=[jax.ShapeDtypeStruct((B, 8), jnp.float32),
                   jax.ShapeDtypeStruct((B, K), jnp.int32),
                   accshape, accshape],
        interpret=_INTERPRET,
    )(x1, Wr[1], br[1][None, :], W1[1], b1[1], W2[1], b2[1],
      H1, bh1[None, :], H2, bh2[None, :])

    inv_b = 1.0 / B
    aux0 = E * jnp.sum(imp0[0, 0] * inv_b * (cnt0[0, 0] * inv_b))
    aux1 = E * jnp.sum(imp1[0, 0] * inv_b * (cnt1[0, 0] * inv_b))
    return (res, topi, aux0 + aux1)


# speedup vs baseline: 1.5286x; 1.5286x over previous
import jax
import jax.numpy as jnp
from jax.experimental import pallas as pl
from jax.experimental.pallas import tpu as pltpu

B = 4096
D = 256
E = 16
K = 4
DFF = 512
BLK = 512
NBLK = B // BLK

_INTERPRET = False


def _dot(a, b):
    return jnp.dot(a, b, preferred_element_type=jnp.float32)


def _topk_gates(logits):
    l = logits
    iota = jax.lax.broadcasted_iota(jnp.int32, l.shape, 1)
    tvs, tis = [], []
    for _ in range(K):
        m = jnp.max(l, axis=1, keepdims=True)
        idx = jnp.min(jnp.where(l == m, iota, E), axis=1, keepdims=True)
        tvs.append(m)
        tis.append(idx)
        l = jnp.where(iota == idx, -jnp.inf, l)
    topv = jnp.concatenate(tvs, axis=1)
    topi = jnp.concatenate(tis, axis=1)
    g = jnp.exp(topv - topv[:, 0:1])
    gates = g / jnp.sum(g, axis=1, keepdims=True)
    comb = jnp.zeros_like(logits)
    cnt = jnp.zeros_like(logits)
    for k in range(K):
        sel = iota == tis[k]
        comb = comb + jnp.where(sel, gates[:, k:k + 1], 0.0)
        cnt = cnt + jnp.where(sel, 1.0, 0.0)
    return topi, comb, cnt


def _router_stats(logits, comb_cnt, imp_ref, cnt_ref):
    mx = jnp.max(logits, axis=1, keepdims=True)
    ex = jnp.exp(logits - mx)
    sm = ex / jnp.sum(ex, axis=1, keepdims=True)

    @pl.when(pl.program_id(0) == 0)
    def _():
        imp_ref[0, 0, :] = jnp.zeros((E,), jnp.float32)
        cnt_ref[0, 0, :] = jnp.zeros((E,), jnp.float32)

    imp_ref[0, 0, :] += jnp.sum(sm, axis=0)
    cnt_ref[0, 0, :] += jnp.sum(comb_cnt, axis=0)


def _ffn_dense(x, comb, W1_ref, b1_ref, W2_ref, b2_ref):
    out = jnp.zeros_like(x)
    for e in range(E):
        h = jnp.maximum(_dot(x, W1_ref[e]) + b1_ref[e:e + 1, :], 0.0)
        y = _dot(h, W2_ref[e]) + b2_ref[e:e + 1, :]
        out = out + comb[:, e:e + 1] * y
    return x + out


def _stage0_body(opi_ref, a_ref, b_ref, c_ref, opt_ref, Wp_ref, bp_ref,
                 Wr_ref, br_ref, W1_ref, b1_ref, W2_ref, b2_ref,
                 x1_ref, imp_ref, cnt_ref):
    op = opi_ref[0, 0, :][:, None]
    av = a_ref[0, 0, :][:, None]
    bv = b_ref[0, 0, :][:, None]
    cv = c_ref[0, 0, :][:, None]
    i8 = jax.lax.broadcasted_iota(jnp.int32, (BLK, 8), 1)
    abits = ((av >> i8) & 1).astype(jnp.float32)
    bbits = ((bv >> i8) & 1).astype(jnp.float32)
    cf = cv.astype(jnp.float32)
    op_emb = jnp.zeros((BLK, 32), jnp.float32)
    for j in range(8):
        op_emb = jnp.where(op == j, opt_ref[j:j + 1, :], op_emb)
    feat = jnp.concatenate(
        [op_emb, abits, bbits, cf, jnp.zeros((BLK, 128 - 49), jnp.float32)], axis=1)
    x = _dot(feat, Wp_ref[...]) + bp_ref[...]
    logits = _dot(x, Wr_ref[...]) + br_ref[...]
    _, comb, cnt_rows = _topk_gates(logits)
    _router_stats(logits, cnt_rows, imp_ref, cnt_ref)
    x1_ref[...] = _ffn_dense(x, comb, W1_ref, b1_ref, W2_ref, b2_ref)


def _stage1_body(x_ref, Wr_ref, br_ref, W1_ref, b1_ref, W2_ref, b2_ref,
                 H1_ref, bh1_ref, H2_ref, bh2_ref,
                 res_ref, topi_ref, imp_ref, cnt_ref):
    x = x_ref[...]
    logits = _dot(x, Wr_ref[...]) + br_ref[...]
    topi, comb, cnt_rows = _topk_gates(logits)
    _router_stats(logits, cnt_rows, imp_ref, cnt_ref)
    x2 = _ffn_dense(x, comb, W1_ref, b1_ref, W2_ref, b2_ref)
    h = jnp.maximum(_dot(x2, H1_ref[...]) + bh1_ref[...], 0.0)
    z = _dot(h, H2_ref[...]) + bh2_ref[...]
    res_ref[...] = 1.0 / (1.0 + jnp.exp(-z))
    topi_ref[...] = topi


def _full(shape):
    nd = len(shape)
    return pl.BlockSpec(shape, lambda i: (0,) * nd)


def kernel(op_idx, a, b, c, op_table, Wp, bp, Wr, br, W1, b1, W2, b2, H1, bh1, H2, bh2):
    tok3 = lambda v: v.reshape(NBLK, 1, BLK)
    tokspec = pl.BlockSpec((1, 1, BLK), lambda i: (i, 0, 0))
    accspec = pl.BlockSpec((1, 1, E), lambda i: (0, 0, 0))
    accshape = jax.ShapeDtypeStruct((1, 1, E), jnp.float32)

    x1, imp0, cnt0 = pl.pallas_call(
        _stage0_body,
        grid=(NBLK,),
        in_specs=[tokspec, tokspec, tokspec, tokspec,
                  _full((8, 32)), _full((128, D)), _full((1, D)),
                  _full((D, E)), _full((1, E)),
                  _full((E, D, DFF)), _full((E, DFF)),
                  _full((E, DFF, D)), _full((E, D))],
        out_specs=[pl.BlockSpec((BLK, D), lambda i: (i, 0)), accspec, accspec],
        out_shape=[jax.ShapeDtypeStruct((B, D), jnp.float32), accshape, accshape],
        interpret=_INTERPRET,
    )(tok3(op_idx), tok3(a), tok3(b), tok3(c), op_table,
      jnp.pad(Wp, ((0, 128 - 49), (0, 0))), bp[None, :],
      Wr[0], br[0][None, :], W1[0], b1[0], W2[0], b2[0])

    res, topi, imp1, cnt1 = pl.pallas_call(
        _stage1_body,
        grid=(NBLK,),
        in_specs=[pl.BlockSpec((BLK, D), lambda i: (i, 0)),
                  _full((D, E)), _full((1, E)),
                  _full((E, D, DFF)), _full((E, DFF)),
                  _full((E, DFF, D)), _full((E, D)),
                  _full((D, 64)), _full((1, 64)), _full((64, 8)), _full((1, 8))],
        out_specs=[pl.BlockSpec((BLK, 8), lambda i: (i, 0)),
                   pl.BlockSpec((BLK, K), lambda i: (i, 0)),
                   accspec, accspec],
        out_shape=[jax.ShapeDtypeStruct((B, 8), jnp.float32),
                   jax.ShapeDtypeStruct((B, K), jnp.int32),
                   accshape, accshape],
        interpret=_INTERPRET,
    )(x1, Wr[1], br[1][None, :], W1[1], b1[1], W2[1], b2[1],
      H1, bh1[None, :], H2, bh2[None, :])

    inv_b = 1.0 / B
    aux0 = E * jnp.sum(imp0[0, 0] * inv_b * (cnt0[0, 0] * inv_b))
    aux1 = E * jnp.sum(imp1[0, 0] * inv_b * (cnt1[0, 0] * inv_b))
    return (res, topi, aux0 + aux1)
